# baseline (speedup 1.0000x reference)
import functools

import jax
import jax.numpy as jnp
from jax import lax
from jax.experimental import pallas as pl
from jax.experimental.pallas import tpu as pltpu
from jax.experimental.pallas import tpu_sc as plsc

VOCAB = 1000000
DIM = 32
NVALS = 81920
PDIM = 128

_info = plsc.get_sparse_core_info()
_NC, _NS = _info.num_cores, _info.num_subcores
_NW = _NC * _NS
_BPW = NVALS // _NW
_CHUNK = 640
_NCHUNK = _BPW // _CHUNK


_mesh = plsc.VectorSubcoreMesh(core_axis_name="c", subcore_axis_name="s")


@functools.partial(
    pl.kernel,
    mesh=_mesh,
    out_type=jax.ShapeDtypeStruct((NVALS, PDIM), jnp.float32),
    scratch_types=[
        pltpu.VMEM((_BPW,), jnp.int32),
        pltpu.VMEM((_CHUNK, PDIM), jnp.float32),
        pltpu.SemaphoreType.DMA,
    ],
)
def _gather1(vals, tab, out, idx_v, rows_v, sem):
    wid = lax.axis_index("s") * _NC + lax.axis_index("c")
    base = wid * _BPW
    pltpu.sync_copy(vals.at[pl.ds(base, _BPW)], idx_v)
    for k in range(_NCHUNK):
        pltpu.async_copy(tab.at[idx_v.at[pl.ds(k * _CHUNK, _CHUNK)]],
                         rows_v, sem).wait()
        pltpu.sync_copy(rows_v, out.at[pl.ds(base + k * _CHUNK, _CHUNK)])


def _lookup(vals, tab):
    padded = jnp.pad(tab, ((0, 0), (0, PDIM - DIM)))
    return _gather1(vals, padded)[:, :DIM]


def kernel(values_f1, lengths_f1, values_f2, lengths_f2,
           values_f3, lengths_f3, values_f4, lengths_f4,
           table_f1, table_f2, table_f3, table_f4):
    o1 = _lookup(values_f1, table_f1)
    o2 = _lookup(values_f2, table_f2)
    o3 = _lookup(values_f3, table_f3)
    o4 = _lookup(values_f4, table_f4)
    return (o1, lengths_f1, o2, lengths_f2, o3, lengths_f3, o4, lengths_f4)


# speedup vs baseline: 1.1346x; 1.1346x over previous
import functools

import jax
import jax.numpy as jnp
from jax import lax
from jax.experimental import pallas as pl
from jax.experimental.pallas import tpu as pltpu
from jax.experimental.pallas import tpu_sc as plsc

VOCAB = 1000000
DIM = 32
NVALS = 81920
CDIM = 4 * DIM

_info = plsc.get_sparse_core_info()
_NC, _NS = _info.num_cores, _info.num_subcores
_NW = _NC * _NS
_BPW = NVALS // _NW
_CHUNK = 640
_NCHUNK = _BPW // _CHUNK


_mesh = plsc.VectorSubcoreMesh(core_axis_name="c", subcore_axis_name="s")


@functools.partial(
    pl.kernel,
    mesh=_mesh,
    out_type=[jax.ShapeDtypeStruct((NVALS, CDIM), jnp.float32)] * 4,
    scratch_types=[
        pltpu.VMEM((_BPW,), jnp.int32),
        pltpu.VMEM((_CHUNK, CDIM), jnp.float32),
        pltpu.SemaphoreType.DMA,
    ],
)
def _gather4(v1, v2, v3, v4, tab, o1, o2, o3, o4, idx_v, rows_v, sem):
    wid = lax.axis_index("s") * _NC + lax.axis_index("c")
    base = wid * _BPW
    for vals, out in ((v1, o1), (v2, o2), (v3, o3), (v4, o4)):
        pltpu.sync_copy(vals.at[pl.ds(base, _BPW)], idx_v)
        for k in range(_NCHUNK):
            pltpu.async_copy(tab.at[idx_v.at[pl.ds(k * _CHUNK, _CHUNK)]],
                             rows_v, sem).wait()
            pltpu.sync_copy(rows_v,
                            out.at[pl.ds(base + k * _CHUNK, _CHUNK)])


def kernel(values_f1, lengths_f1, values_f2, lengths_f2,
           values_f3, lengths_f3, values_f4, lengths_f4,
           table_f1, table_f2, table_f3, table_f4):
    tab = jnp.concatenate([table_f1, table_f2, table_f3, table_f4], axis=1)
    o1, o2, o3, o4 = _gather4(values_f1, values_f2, values_f3, values_f4, tab)
    return (o1[:, 0:DIM], lengths_f1,
            o2[:, DIM:2 * DIM], lengths_f2,
            o3[:, 2 * DIM:3 * DIM], lengths_f3,
            o4[:, 3 * DIM:4 * DIM], lengths_f4)


# speedup vs baseline: 1.1373x; 1.0023x over previous
import functools

import jax
import jax.numpy as jnp
from jax import lax
from jax.experimental import pallas as pl
from jax.experimental.pallas import tpu as pltpu
from jax.experimental.pallas import tpu_sc as plsc

VOCAB = 1000000
DIM = 32
NVALS = 81920
CDIM = 4 * DIM

_info = plsc.get_sparse_core_info()
_NC, _NS = _info.num_cores, _info.num_subcores
_NW = _NC * _NS
_BPW = NVALS // _NW
_CHUNK = 320
_NCHUNK = _BPW // _CHUNK


_mesh = plsc.VectorSubcoreMesh(core_axis_name="c", subcore_axis_name="s")


@functools.partial(
    pl.kernel,
    mesh=_mesh,
    out_type=[jax.ShapeDtypeStruct((NVALS, CDIM), jnp.float32)] * 4,
    scratch_types=[
        pltpu.VMEM((_BPW,), jnp.int32),
        pltpu.VMEM((_CHUNK, CDIM), jnp.float32),
        pltpu.VMEM((_CHUNK, CDIM), jnp.float32),
        pltpu.SemaphoreType.DMA,
    ],
)
def _gather4(v1, v2, v3, v4, tab, o1, o2, o3, o4,
             idx_v, rows_a, rows_b, sem):
    wid = lax.axis_index("s") * _NC + lax.axis_index("c")
    base = wid * _BPW
    bufs = (rows_a, rows_b)
    for vals, out in ((v1, o1), (v2, o2), (v3, o3), (v4, o4)):
        pltpu.sync_copy(vals.at[pl.ds(base, _BPW)], idx_v)
        pltpu.async_copy(tab.at[idx_v.at[pl.ds(0, _CHUNK)]], bufs[0], sem)
        for k in range(_NCHUNK):
            buf = bufs[k % 2]
            pltpu.make_async_copy(tab.at[pl.ds(0, _CHUNK)], buf, sem).wait()
            if k + 1 < _NCHUNK:
                pltpu.async_copy(
                    tab.at[idx_v.at[pl.ds((k + 1) * _CHUNK, _CHUNK)]],
                    bufs[(k + 1) % 2], sem)
            pltpu.sync_copy(buf, out.at[pl.ds(base + k * _CHUNK, _CHUNK)])


def kernel(values_f1, lengths_f1, values_f2, lengths_f2,
           values_f3, lengths_f3, values_f4, lengths_f4,
           table_f1, table_f2, table_f3, table_f4):
    tab = jnp.concatenate([table_f1, table_f2, table_f3, table_f4], axis=1)
    o1, o2, o3, o4 = _gather4(values_f1, values_f2, values_f3, values_f4, tab)
    return (o1[:, 0:DIM], lengths_f1,
            o2[:, DIM:2 * DIM], lengths_f2,
            o3[:, 2 * DIM:3 * DIM], lengths_f3,
            o4[:, 3 * DIM:4 * DIM], lengths_f4)
